# baseline (speedup 1.0000x reference)
import numpy as np
import jax
import jax.numpy as jnp
from jax import lax
from jax.experimental import pallas as pl
from jax.experimental.pallas import tpu as pltpu
from jax.experimental.pallas import tpu_sc as plsc

_ANCHORS = np.array([
    [[0.28, 0.22], [0.38, 0.48], [0.90, 0.78]],
    [[0.07, 0.15], [0.15, 0.11], [0.14, 0.29]],
    [[0.02, 0.03], [0.04, 0.07], [0.08, 0.06]],
], dtype=np.float32)
_S_LIST = [13, 26, 52]
_BATCH = 32
_NC, _NS = 2, 16
_NW = _NC * _NS

_N_CELLS = [_BATCH * 3 * s * s for s in _S_LIST]
_CH = [512, 2032, 8112]
_SS = [s * s for s in _S_LIST]

_anch_tab = np.zeros(32, np.float32)
for _i in range(3):
    for _k in range(3):
        _anch_tab[_i * 8 + 2 * _k] = 1.0 / (_ANCHORS[_i, _k, 0] * _S_LIST[_i])
        _anch_tab[_i * 8 + 2 * _k + 1] = 1.0 / (_ANCHORS[_i, _k, 1] * _S_LIST[_i])

_LN2 = 0.6931471805599453
_SQRT2 = 1.4142135623730951


def _softlog(x):
    b = plsc.bitcast(x, jnp.int32)
    e = lax.shift_right_logical(b, 23) - 127
    m = plsc.bitcast((b & 0x007FFFFF) | 0x3F800000, jnp.float32)
    big = m >= _SQRT2
    m = jnp.where(big, m * 0.5, m)
    e = e + jnp.where(big, 1, 0)
    t = (m - 1.0) / (m + 1.0)
    t2 = t * t
    p = 1.0 + t2 * (1.0 / 3.0 + t2 * (0.2 + t2 * (1.0 / 7.0 + t2 * (1.0 / 9.0))))
    return e.astype(jnp.float32) * _LN2 + 2.0 * t * p


def _do_scale(i, pred_ref, tgt_ref, out_ref, tgt_buf, idx_cell, idx_ebuf,
              gath_buf, anch_v, stage, sem, wid):
    n_cells, ch, ss = _N_CELLS[i], _CH[i], _SS[i]
    lane = lax.iota(jnp.int32, 16)
    base_owned = wid * ch
    base_read = jnp.minimum(base_owned, n_cells - ch)
    skip = base_owned - base_read

    pltpu.sync_copy(tgt_ref.at[pl.ds(base_read * 6, ch * 6)],
                    tgt_buf.at[pl.ds(0, ch * 6)])

    def scan_body(j, m_vec):
        lidx = j * 16 + lane
        t0 = plsc.load_gather(tgt_buf, [lidx * 6])
        msk = (t0 == 1.0) & (lidx >= skip)
        pos = m_vec + plsc.cumsum(msk.astype(jnp.int32)) - 1
        plsc.store_scatter(idx_cell, [pos], lidx, mask=msk)
        return m_vec + plsc.all_reduce_population_count(msk)

    m_vec = lax.fori_loop(0, ch // 16, scan_body, jnp.zeros(16, jnp.int32))
    m = jnp.max(m_vec)
    nch = (m + 31) // 32

    def chunk_body(ci, acc):
        cells = []
        for v in range(2):
            cpos = ci * 32 + v * 16 + lane
            vld = cpos < m
            cell = plsc.load_gather(idx_cell, [cpos])
            cell = jnp.where(vld, cell, 0)
            cells.append((cell, vld))
            fb = (base_read + cell) * 4
            for c in range(4):
                idx_ebuf[pl.ds(c * 32 + v * 16, 16)] = fb + c
        pltpu.async_copy(pred_ref.at[idx_ebuf], gath_buf, sem).wait()
        for v in range(2):
            cell, vld = cells[v]
            tb = cell * 6
            tx = plsc.load_gather(tgt_buf, [tb + 1])
            ty = plsc.load_gather(tgt_buf, [tb + 2])
            tw = plsc.load_gather(tgt_buf, [tb + 3])
            th = plsc.load_gather(tgt_buf, [tb + 4])
            px = gath_buf[pl.ds(0 * 32 + v * 16, 16)]
            py = gath_buf[pl.ds(1 * 32 + v * 16, 16)]
            pw = gath_buf[pl.ds(2 * 32 + v * 16, 16)]
            ph = gath_buf[pl.ds(3 * 32 + v * 16, 16)]
            k = ((base_read + cell) // ss) % 3
            iw = plsc.load_gather(anch_v, [i * 8 + 2 * k])
            ih = plsc.load_gather(anch_v, [i * 8 + 2 * k + 1])
            sx = 1.0 / (1.0 + jnp.exp(-px))
            sy = 1.0 / (1.0 + jnp.exp(-py))
            lw = _softlog(1e-16 + tw * iw)
            lh = _softlog(1e-16 + th * ih)
            dx, dy, dw, dh = sx - tx, sy - ty, pw - lw, ph - lh
            d = dx * dx + dy * dy + dw * dw + dh * dh
            acc = acc + jnp.where(vld, d, 0.0)
        return acc

    acc = lax.fori_loop(0, nch, chunk_body, jnp.zeros(16, jnp.float32))

    stage[...] = acc
    pltpu.sync_copy(stage, out_ref.at[wid, i])
    stage[...] = m_vec.astype(jnp.float32)
    pltpu.sync_copy(stage, out_ref.at[wid, 3 + i])


def _body(p0, p1, p2, t0, t1, t2, anch, out_ref, tgt_buf, idx_cell, idx_ebuf,
          gath_buf, anch_v, stage, sem):
    wid = lax.axis_index("s") * _NC + lax.axis_index("c")
    pltpu.sync_copy(anch, anch_v)
    preds = [p0, p1, p2]
    tgts = [t0, t1, t2]
    for i in range(3):
        _do_scale(i, preds[i], tgts[i], out_ref, tgt_buf, idx_cell, idx_ebuf,
                  gath_buf, anch_v, stage, sem, wid)


_sc_call = pl.kernel(
    _body,
    out_type=jax.ShapeDtypeStruct((_NW, 8, 16), jnp.float32),
    mesh=plsc.VectorSubcoreMesh(core_axis_name="c", subcore_axis_name="s",
                                num_cores=_NC, num_subcores=_NS),
    compiler_params=pltpu.CompilerParams(needs_layout_passes=False),
    scratch_types=[
        pltpu.VMEM((_CH[2] * 6,), jnp.float32),
        pltpu.VMEM((8192,), jnp.int32),
        pltpu.VMEM((128,), jnp.int32),
        pltpu.VMEM((128,), jnp.float32),
        pltpu.VMEM((32,), jnp.float32),
        pltpu.VMEM((16,), jnp.float32),
        pltpu.SemaphoreType.DMA,
    ],
)


_RB = 32


def _extract_body(in_ref, out_ref):
    out_ref[...] = in_ref[:, :, 1:5]


def _extract_box(x, s):
    r = x.shape[0]
    return pl.pallas_call(
        _extract_body,
        grid=(r // _RB,),
        in_specs=[pl.BlockSpec((_RB, s, 85), lambda i: (i, 0, 0))],
        out_specs=pl.BlockSpec((_RB, s, 4), lambda i: (i, 0, 0)),
        out_shape=jax.ShapeDtypeStruct((r, s, 4), jnp.float32),
    )(x)


def kernel(pred_0, pred_1, pred_2, target_0, target_1, target_2):
    p = [_extract_box(x.reshape(-1, s, 85), s).reshape(-1)
         for x, s in zip((pred_0, pred_1, pred_2), _S_LIST)]
    t = [x.reshape(-1) for x in (target_0, target_1, target_2)]
    anch = jnp.asarray(_anch_tab)
    parts = _sc_call(p[0], p[1], p[2], t[0], t[1], t[2], anch)
    s = parts[:, 0:3, :].sum(axis=(0, 2))
    cnt = parts[:, 3:6, 0].sum(axis=0)
    return (10.0 * s / jnp.maximum(4.0 * cnt, 1.0)).sum()


# speedup vs baseline: 13.1093x; 13.1093x over previous
import numpy as np
import jax
import jax.numpy as jnp
from jax import lax
from jax.experimental import pallas as pl
from jax.experimental.pallas import tpu as pltpu
from jax.experimental.pallas import tpu_sc as plsc

_ANCHORS = np.array([
    [[0.28, 0.22], [0.38, 0.48], [0.90, 0.78]],
    [[0.07, 0.15], [0.15, 0.11], [0.14, 0.29]],
    [[0.02, 0.03], [0.04, 0.07], [0.08, 0.06]],
], dtype=np.float32)
_S_LIST = [13, 26, 52]
_BATCH = 32
_NC, _NS = 2, 16
_NW = _NC * _NS

_NBLK = [3 * s for s in _S_LIST]
_NPW = [(n + _NW - 1) // _NW for n in _NBLK]
_CB = [_BATCH * s for s in _S_LIST]

_anch_tab = np.zeros(32, np.float32)
for _i in range(3):
    for _k in range(3):
        _anch_tab[_i * 8 + 2 * _k] = 1.0 / (_ANCHORS[_i, _k, 0] * _S_LIST[_i])
        _anch_tab[_i * 8 + 2 * _k + 1] = 1.0 / (_ANCHORS[_i, _k, 1] * _S_LIST[_i])

_LN2 = 0.6931471805599453
_SQRT2 = 1.4142135623730951


def _softlog(x):
    b = plsc.bitcast(x, jnp.int32)
    e = lax.shift_right_logical(b, 23) - 127
    m = plsc.bitcast((b & 0x007FFFFF) | 0x3F800000, jnp.float32)
    big = m >= _SQRT2
    m = jnp.where(big, m * 0.5, m)
    e = e + jnp.where(big, 1, 0)
    t = (m - 1.0) / (m + 1.0)
    t2 = t * t
    p = 1.0 + t2 * (1.0 / 3.0 + t2 * (0.2 + t2 * (1.0 / 7.0 + t2 * (1.0 / 9.0))))
    return e.astype(jnp.float32) * _LN2 + 2.0 * t * p


def _do_scale(i, pred_ref, tgt_ref, out_ref, planes, predb, idx_cell, anch_v,
              stage, tsem, psem, wid):
    s = _S_LIST[i]
    nblk, cb = _NBLK[i], _CB[i]
    yb_order = (i == 0)
    lane = lax.iota(jnp.int32, 16)
    acc = jnp.zeros(16, jnp.float32)
    macc = jnp.zeros(16, jnp.int32)

    for t in range(_NPW[i]):
        blk_real = wid + t * _NW
        blk_ok = blk_real < nblk
        blk = jnp.minimum(blk_real, nblk - 1)

        copies = [pltpu.async_copy(tgt_ref.at[blk, c], planes[c], tsem)
                  for c in range(5)]
        for c in copies:
            c.wait()

        def scan_body(j, m_vec, blk_ok=blk_ok):
            lidx = j * 16 + lane
            if yb_order:
                yv = lax.shift_right_logical(lidx, 5)
                bv = lidx & 31
                t0 = plsc.load_gather(planes[0], [yv, bv])
            else:
                bv = lidx // s
                yv = lidx - bv * s
                t0 = plsc.load_gather(planes[0], [bv, yv])
            msk = (t0 == 1.0) & blk_ok
            pos = m_vec + plsc.cumsum(msk.astype(jnp.int32)) - 1
            plsc.store_scatter(idx_cell, [pos], lidx, mask=msk)
            return m_vec + plsc.all_reduce_population_count(msk)

        m_vec = lax.fori_loop(0, cb // 16, scan_body, jnp.zeros(16, jnp.int32))
        macc = macc + m_vec
        m = jnp.max(m_vec)
        nb16 = (m + 15) // 16

        def batch_body(v, acc, blk=blk):
            cpos = v * 16 + lane
            vld = cpos < m
            cell = plsc.load_gather(idx_cell, [cpos])
            cell = jnp.where(vld, cell, 0)
            if yb_order:
                yv = lax.shift_right_logical(cell, 5)
                bv = cell & 31
            else:
                bv = cell // s
                yv = cell - bv * s
            for jj in range(16):
                l_s = cell[jj]
                if yb_order:
                    y_s = lax.shift_right_logical(l_s, 5)
                    b_s = l_s & 31
                else:
                    b_s = l_s // s
                    y_s = l_s - b_s * s
                b8 = pl.multiple_of(b_s & ~7, 8)
                pltpu.async_copy(pred_ref.at[blk * s + y_s, pl.ds(b8, 8)],
                                 predb.at[pl.ds(jj * 8, 8)], psem)
            for _ in range(16):
                pltpu.make_async_copy(pred_ref.at[0, pl.ds(0, 8)],
                                      predb.at[pl.ds(0, 8)], psem).wait()
            if yb_order:
                tx = plsc.load_gather(planes[1], [yv, bv])
                ty = plsc.load_gather(planes[2], [yv, bv])
                tw = plsc.load_gather(planes[3], [yv, bv])
                th = plsc.load_gather(planes[4], [yv, bv])
            else:
                tx = plsc.load_gather(planes[1], [bv, yv])
                ty = plsc.load_gather(planes[2], [bv, yv])
                tw = plsc.load_gather(planes[3], [bv, yv])
                th = plsc.load_gather(planes[4], [bv, yv])
            prow = lane * 8 + (bv & 7)
            px = plsc.load_gather(predb, [prow, lane * 0 + 1])
            py = plsc.load_gather(predb, [prow, lane * 0 + 2])
            pw = plsc.load_gather(predb, [prow, lane * 0 + 3])
            ph = plsc.load_gather(predb, [prow, lane * 0 + 4])
            a_sc = blk // s
            iw = plsc.load_gather(anch_v, [lane * 0 + (i * 8 + 2 * a_sc)])
            ih = plsc.load_gather(anch_v, [lane * 0 + (i * 8 + 2 * a_sc + 1)])
            sx = 1.0 / (1.0 + jnp.exp(-px))
            sy = 1.0 / (1.0 + jnp.exp(-py))
            lw = _softlog(1e-16 + tw * iw)
            lh = _softlog(1e-16 + th * ih)
            dx, dy, dw, dh = sx - tx, sy - ty, pw - lw, ph - lh
            d = dx * dx + dy * dy + dw * dw + dh * dh
            return acc + jnp.where(vld, d, 0.0)

        acc = lax.fori_loop(0, nb16, batch_body, acc)

    stage[...] = acc
    pltpu.sync_copy(stage, out_ref.at[wid, i])
    stage[...] = macc.astype(jnp.float32)
    pltpu.sync_copy(stage, out_ref.at[wid, 3 + i])


def _body(p0, p1, p2, t0, t1, t2, anch, out_ref,
          pa0, pb0, pc0, pd0, pe0,
          pa1, pb1, pc1, pd1, pe1,
          pa2, pb2, pc2, pd2, pe2,
          predb, idx_cell, anch_v, stage, tsem, psem):
    wid = lax.axis_index("s") * _NC + lax.axis_index("c")
    pltpu.sync_copy(anch, anch_v)
    planes = [[pa0, pb0, pc0, pd0, pe0],
              [pa1, pb1, pc1, pd1, pe1],
              [pa2, pb2, pc2, pd2, pe2]]
    preds = [p0, p1, p2]
    tgts = [t0, t1, t2]
    for i in range(3):
        _do_scale(i, preds[i], tgts[i], out_ref, planes[i], predb, idx_cell,
                  anch_v, stage, tsem, psem, wid)


_plane_shapes = [(13, 32), (32, 26), (32, 52)]

_sc_call = pl.kernel(
    _body,
    out_type=jax.ShapeDtypeStruct((_NW, 8, 16), jnp.float32),
    mesh=plsc.VectorSubcoreMesh(core_axis_name="c", subcore_axis_name="s",
                                num_cores=_NC, num_subcores=_NS),
    compiler_params=pltpu.CompilerParams(needs_layout_passes=False),
    scratch_types=(
        [pltpu.VMEM(_plane_shapes[0], jnp.float32) for _ in range(5)]
        + [pltpu.VMEM(_plane_shapes[1], jnp.float32) for _ in range(5)]
        + [pltpu.VMEM(_plane_shapes[2], jnp.float32) for _ in range(5)]
        + [
            pltpu.VMEM((128, 85), jnp.float32),
            pltpu.VMEM((2048,), jnp.int32),
            pltpu.VMEM((32,), jnp.float32),
            pltpu.VMEM((16,), jnp.float32),
            pltpu.SemaphoreType.DMA,
            pltpu.SemaphoreType.DMA,
        ]
    ),
)


def kernel(pred_0, pred_1, pred_2, target_0, target_1, target_2):
    p = [x.transpose(1, 2, 3, 0, 4).reshape(-1, _BATCH, 85)
         for x in (pred_0, pred_1, pred_2)]
    t0 = target_0.transpose(1, 2, 4, 3, 0).reshape(-1, 6, 13, 32)
    t1 = target_1.transpose(1, 2, 4, 0, 3).reshape(-1, 6, 32, 26)
    t2 = target_2.transpose(1, 2, 4, 0, 3).reshape(-1, 6, 32, 52)
    anch = jnp.asarray(_anch_tab)
    parts = _sc_call(p[0], p[1], p[2], t0, t1, t2, anch)
    s = parts[:, 0:3, :].sum(axis=(0, 2))
    cnt = parts[:, 3:6, 0].sum(axis=0)
    return (10.0 * s / jnp.maximum(4.0 * cnt, 1.0)).sum()


# speedup vs baseline: 13.7227x; 1.0468x over previous
import numpy as np
import jax
import jax.numpy as jnp
from jax import lax
from jax.experimental import pallas as pl
from jax.experimental.pallas import tpu as pltpu
from jax.experimental.pallas import tpu_sc as plsc

_ANCHORS = np.array([
    [[0.28, 0.22], [0.38, 0.48], [0.90, 0.78]],
    [[0.07, 0.15], [0.15, 0.11], [0.14, 0.29]],
    [[0.02, 0.03], [0.04, 0.07], [0.08, 0.06]],
], dtype=np.float32)
_S_LIST = [13, 26, 52]
_BATCH = 32
_NC, _NS = 2, 16
_NW = _NC * _NS

_NBLK = [3 * s for s in _S_LIST]
_NPW = [(n + _NW - 1) // _NW for n in _NBLK]
_CB = [_BATCH * s for s in _S_LIST]
_NSET = [1, 1, 2]

_anch_tab = np.zeros(32, np.float32)
for _i in range(3):
    for _k in range(3):
        _anch_tab[_i * 8 + 2 * _k] = 1.0 / (_ANCHORS[_i, _k, 0] * _S_LIST[_i])
        _anch_tab[_i * 8 + 2 * _k + 1] = 1.0 / (_ANCHORS[_i, _k, 1] * _S_LIST[_i])

_LN2 = 0.6931471805599453
_SQRT2 = 1.4142135623730951


def _softlog(x):
    b = plsc.bitcast(x, jnp.int32)
    e = lax.shift_right_logical(b, 23) - 127
    m = plsc.bitcast((b & 0x007FFFFF) | 0x3F800000, jnp.float32)
    big = m >= _SQRT2
    m = jnp.where(big, m * 0.5, m)
    e = e + jnp.where(big, 1, 0)
    t = (m - 1.0) / (m + 1.0)
    t2 = t * t
    p = 1.0 + t2 * (1.0 / 3.0 + t2 * (0.2 + t2 * (1.0 / 7.0 + t2 * (1.0 / 9.0))))
    return e.astype(jnp.float32) * _LN2 + 2.0 * t * p


def _do_scale(i, pred_ref, tgt_ref, out_ref, plane_sets, tsems, predb,
              idx_cell, anch_v, stage, psem, wid):
    s = _S_LIST[i]
    nblk, cb = _NBLK[i], _CB[i]
    nset = len(plane_sets)
    yb_order = (i == 0)
    lane = lax.iota(jnp.int32, 16)
    acc = jnp.zeros(16, jnp.float32)
    macc = jnp.zeros(16, jnp.int32)

    def issue_planes(t):
        blk = jnp.minimum(wid + t * _NW, nblk - 1)
        st = t % nset
        return [pltpu.async_copy(tgt_ref.at[blk, c], plane_sets[st][c],
                                 tsems[st]) for c in range(5)]

    pending = issue_planes(0)
    for t in range(_NPW[i]):
        planes = plane_sets[t % nset]
        blk_real = wid + t * _NW
        blk_ok = blk_real < nblk
        blk = jnp.minimum(blk_real, nblk - 1)

        for cp in pending:
            cp.wait()
        if nset > 1 and t + 1 < _NPW[i]:
            pending = issue_planes(t + 1)

        def scan_body(j, m_vec, blk_ok=blk_ok, planes=planes):
            lidx = j * 16 + lane
            if yb_order:
                yv = lax.shift_right_logical(lidx, 5)
                bv = lidx & 31
                t0 = plsc.load_gather(planes[0], [yv, bv])
            else:
                bv = lidx // s
                yv = lidx - bv * s
                t0 = plsc.load_gather(planes[0], [bv, yv])
            msk = (t0 == 1.0) & blk_ok
            pos = m_vec + plsc.cumsum(msk.astype(jnp.int32)) - 1
            plsc.store_scatter(idx_cell, [pos], lidx, mask=msk)
            return m_vec + plsc.all_reduce_population_count(msk)

        m_vec = lax.fori_loop(0, cb // 16, scan_body, jnp.zeros(16, jnp.int32))
        macc = macc + m_vec
        m = jnp.max(m_vec)
        nb32 = (m + 31) // 32

        def batch_body(v, acc, blk=blk, planes=planes):
            cells = []
            for w in range(2):
                cpos = v * 32 + w * 16 + lane
                vld = cpos < m
                cell = plsc.load_gather(idx_cell, [cpos])
                cell = jnp.where(vld, cell, 0)
                cells.append((cell, vld))
                for jj in range(16):
                    l_s = cell[jj]
                    if yb_order:
                        y_s = lax.shift_right_logical(l_s, 5)
                        b_s = l_s & 31
                    else:
                        b_s = l_s // s
                        y_s = l_s - b_s * s
                    b8 = pl.multiple_of(b_s & ~7, 8)
                    pltpu.async_copy(
                        pred_ref.at[blk * s + y_s, pl.ds(b8, 8)],
                        predb.at[pl.ds((w * 16 + jj) * 8, 8)], psem)
            for _ in range(8):
                pltpu.make_async_copy(pred_ref.at[0],
                                      predb.at[pl.ds(0, 32)], psem).wait()
            for w in range(2):
                cell, vld = cells[w]
                if yb_order:
                    yv = lax.shift_right_logical(cell, 5)
                    bv = cell & 31
                    tx = plsc.load_gather(planes[1], [yv, bv])
                    ty = plsc.load_gather(planes[2], [yv, bv])
                    tw = plsc.load_gather(planes[3], [yv, bv])
                    th = plsc.load_gather(planes[4], [yv, bv])
                else:
                    bv = cell // s
                    yv = cell - bv * s
                    tx = plsc.load_gather(planes[1], [bv, yv])
                    ty = plsc.load_gather(planes[2], [bv, yv])
                    tw = plsc.load_gather(planes[3], [bv, yv])
                    th = plsc.load_gather(planes[4], [bv, yv])
                prow = (w * 16 + lane) * 8 + (bv & 7)
                px = plsc.load_gather(predb, [prow, lane * 0 + 1])
                py = plsc.load_gather(predb, [prow, lane * 0 + 2])
                pw = plsc.load_gather(predb, [prow, lane * 0 + 3])
                ph = plsc.load_gather(predb, [prow, lane * 0 + 4])
                a_sc = blk // s
                iw = plsc.load_gather(anch_v, [lane * 0 + (i * 8 + 2 * a_sc)])
                ih = plsc.load_gather(anch_v,
                                      [lane * 0 + (i * 8 + 2 * a_sc + 1)])
                sx = 1.0 / (1.0 + jnp.exp(-px))
                sy = 1.0 / (1.0 + jnp.exp(-py))
                lw = _softlog(1e-16 + tw * iw)
                lh = _softlog(1e-16 + th * ih)
                dx, dy, dw, dh = sx - tx, sy - ty, pw - lw, ph - lh
                d = dx * dx + dy * dy + dw * dw + dh * dh
                acc = acc + jnp.where(vld, d, 0.0)
            return acc

        acc = lax.fori_loop(0, nb32, batch_body, acc)
        if nset == 1 and t + 1 < _NPW[i]:
            pending = issue_planes(t + 1)

    stage[...] = acc
    pltpu.sync_copy(stage, out_ref.at[wid, i])
    stage[...] = macc.astype(jnp.float32)
    pltpu.sync_copy(stage, out_ref.at[wid, 3 + i])


def _body(p0, p1, p2, t0, t1, t2, anch, out_ref,
          pa0, pb0, pc0, pd0, pe0,
          pa1, pb1, pc1, pd1, pe1,
          pa2, pb2, pc2, pd2, pe2,
          qa2, qb2, qc2, qd2, qe2,
          predb, idx_cell, anch_v, stage, tsem0, tsem1, psem):
    wid = lax.axis_index("s") * _NC + lax.axis_index("c")
    pltpu.sync_copy(anch, anch_v)
    plane_sets = [
        [[pa0, pb0, pc0, pd0, pe0]],
        [[pa1, pb1, pc1, pd1, pe1]],
        [[pa2, pb2, pc2, pd2, pe2], [qa2, qb2, qc2, qd2, qe2]],
    ]
    preds = [p0, p1, p2]
    tgts = [t0, t1, t2]
    for i in range(3):
        _do_scale(i, preds[i], tgts[i], out_ref, plane_sets[i],
                  [tsem0, tsem1], predb, idx_cell, anch_v, stage, psem, wid)


_plane_shapes = [(13, 32), (32, 26), (32, 52)]

_sc_call = pl.kernel(
    _body,
    out_type=jax.ShapeDtypeStruct((_NW, 8, 16), jnp.float32),
    mesh=plsc.VectorSubcoreMesh(core_axis_name="c", subcore_axis_name="s",
                                num_cores=_NC, num_subcores=_NS),
    compiler_params=pltpu.CompilerParams(needs_layout_passes=False),
    scratch_types=(
        [pltpu.VMEM(_plane_shapes[0], jnp.float32) for _ in range(5)]
        + [pltpu.VMEM(_plane_shapes[1], jnp.float32) for _ in range(5)]
        + [pltpu.VMEM(_plane_shapes[2], jnp.float32) for _ in range(10)]
        + [
            pltpu.VMEM((256, 85), jnp.float32),
            pltpu.VMEM((2048,), jnp.int32),
            pltpu.VMEM((32,), jnp.float32),
            pltpu.VMEM((16,), jnp.float32),
            pltpu.SemaphoreType.DMA,
            pltpu.SemaphoreType.DMA,
            pltpu.SemaphoreType.DMA,
        ]
    ),
)


def kernel(pred_0, pred_1, pred_2, target_0, target_1, target_2):
    p = [x.transpose(1, 2, 3, 0, 4).reshape(-1, _BATCH, 85)
         for x in (pred_0, pred_1, pred_2)]
    t0 = target_0.transpose(1, 2, 4, 3, 0).reshape(-1, 6, 13, 32)
    t1 = target_1.transpose(1, 2, 4, 0, 3).reshape(-1, 6, 32, 26)
    t2 = target_2.transpose(1, 2, 4, 0, 3).reshape(-1, 6, 32, 52)
    anch = jnp.asarray(_anch_tab)
    parts = _sc_call(p[0], p[1], p[2], t0, t1, t2, anch)
    s = parts[:, 0:3, :].sum(axis=(0, 2))
    cnt = parts[:, 3:6, 0].sum(axis=0)
    return (10.0 * s / jnp.maximum(4.0 * cnt, 1.0)).sum()


# speedup vs baseline: 13.7656x; 1.0031x over previous
import numpy as np
import jax
import jax.numpy as jnp
from jax import lax
from jax.experimental import pallas as pl
from jax.experimental.pallas import tpu as pltpu
from jax.experimental.pallas import tpu_sc as plsc

_ANCHORS = np.array([
    [[0.28, 0.22], [0.38, 0.48], [0.90, 0.78]],
    [[0.07, 0.15], [0.15, 0.11], [0.14, 0.29]],
    [[0.02, 0.03], [0.04, 0.07], [0.08, 0.06]],
], dtype=np.float32)
_S_LIST = [13, 26, 52]
_BATCH = 32
_NC, _NS = 2, 16
_NW = _NC * _NS

_NBLK = [3 * s for s in _S_LIST]
_NPW = [(n + _NW - 1) // _NW for n in _NBLK]
_CB = [_BATCH * s for s in _S_LIST]
_NSET = [1, 1, 2]

_anch_tab = np.zeros(32, np.float32)
for _i in range(3):
    for _k in range(3):
        _anch_tab[_i * 8 + 2 * _k] = 1.0 / (_ANCHORS[_i, _k, 0] * _S_LIST[_i])
        _anch_tab[_i * 8 + 2 * _k + 1] = 1.0 / (_ANCHORS[_i, _k, 1] * _S_LIST[_i])

_LN2 = 0.6931471805599453
_SQRT2 = 1.4142135623730951


def _softlog(x):
    b = plsc.bitcast(x, jnp.int32)
    e = lax.shift_right_logical(b, 23) - 127
    m = plsc.bitcast((b & 0x007FFFFF) | 0x3F800000, jnp.float32)
    big = m >= _SQRT2
    m = jnp.where(big, m * 0.5, m)
    e = e + jnp.where(big, 1, 0)
    t = (m - 1.0) / (m + 1.0)
    t2 = t * t
    p = 1.0 + t2 * (1.0 / 3.0 + t2 * (0.2 + t2 * (1.0 / 7.0 + t2 * (1.0 / 9.0))))
    return e.astype(jnp.float32) * _LN2 + 2.0 * t * p


def _do_scale(i, pred_ref, tgt_ref, out_ref, plane_sets, tsems, predb,
              ebuf, idx_cell, anch_v, stage, psem, wid):
    s = _S_LIST[i]
    nblk, cb = _NBLK[i], _CB[i]
    nset = len(plane_sets)
    yb_order = (i == 0)
    lane = lax.iota(jnp.int32, 16)
    acc = jnp.zeros(16, jnp.float32)
    macc = jnp.zeros(16, jnp.int32)

    def issue_planes(t):
        blk = jnp.minimum(wid + t * _NW, nblk - 1)
        st = t % nset
        return [pltpu.async_copy(tgt_ref.at[blk, c], plane_sets[st][c],
                                 tsems[st]) for c in range(5)]

    pending = issue_planes(0)
    for t in range(_NPW[i]):
        planes = plane_sets[t % nset]
        blk_real = wid + t * _NW
        blk_ok = blk_real < nblk
        blk = jnp.minimum(blk_real, nblk - 1)

        for cp in pending:
            cp.wait()
        if nset > 1 and t + 1 < _NPW[i]:
            pending = issue_planes(t + 1)

        def scan_body(j, m_vec, blk_ok=blk_ok, planes=planes):
            lidx = j * 16 + lane
            if yb_order:
                yv = lax.shift_right_logical(lidx, 5)
                bv = lidx & 31
                t0 = plsc.load_gather(planes[0], [yv, bv])
            else:
                bv = lidx // s
                yv = lidx - bv * s
                t0 = plsc.load_gather(planes[0], [bv, yv])
            msk = (t0 == 1.0) & blk_ok
            pos = m_vec + plsc.cumsum(msk.astype(jnp.int32)) - 1
            plsc.store_scatter(idx_cell, [pos], lidx, mask=msk)
            return m_vec + plsc.all_reduce_population_count(msk)

        m_vec = lax.fori_loop(0, cb // 16, scan_body, jnp.zeros(16, jnp.int32))
        macc = macc + m_vec
        m = jnp.max(m_vec)
        nb32 = (m + 31) // 32

        def batch_body(v, acc, blk=blk, planes=planes):
            cells = []
            for w in range(2):
                cpos = v * 32 + w * 16 + lane
                vld = cpos < m
                cell = plsc.load_gather(idx_cell, [cpos])
                cell = jnp.where(vld, cell, 0)
                cells.append((cell, vld))
                for jj in range(16):
                    l_s = cell[jj]
                    if yb_order:
                        y_s = lax.shift_right_logical(l_s, 5)
                        b_s = l_s & 31
                    else:
                        b_s = l_s // s
                        y_s = l_s - b_s * s
                    b8 = pl.multiple_of(b_s & ~7, 8)
                    pltpu.async_copy(
                        pred_ref.at[blk * s + y_s, pl.ds(b8, 8)],
                        predb.at[pl.ds((w * 16 + jj) * 8, 8)], psem)
            tside = []
            for w in range(2):
                cell, vld = cells[w]
                if yb_order:
                    yv = lax.shift_right_logical(cell, 5)
                    bv = cell & 31
                    tx = plsc.load_gather(planes[1], [yv, bv])
                    ty = plsc.load_gather(planes[2], [yv, bv])
                    tw = plsc.load_gather(planes[3], [yv, bv])
                    th = plsc.load_gather(planes[4], [yv, bv])
                else:
                    bv = cell // s
                    yv = cell - bv * s
                    tx = plsc.load_gather(planes[1], [bv, yv])
                    ty = plsc.load_gather(planes[2], [bv, yv])
                    tw = plsc.load_gather(planes[3], [bv, yv])
                    th = plsc.load_gather(planes[4], [bv, yv])
                a_sc = blk // s
                iw = plsc.load_gather(anch_v, [lane * 0 + (i * 8 + 2 * a_sc)])
                ih = plsc.load_gather(anch_v,
                                      [lane * 0 + (i * 8 + 2 * a_sc + 1)])
                lw = _softlog(1e-16 + tw * iw)
                lh = _softlog(1e-16 + th * ih)
                tside.append((tx, ty, lw, lh, bv))
            for _ in range(8):
                pltpu.make_async_copy(pred_ref.at[0],
                                      predb.at[pl.ds(0, 32)], psem).wait()
            for w in range(2):
                cell, vld = cells[w]
                tx, ty, lw, lh, bv = tside[w]
                prow = (w * 16 + lane) * 8 + (bv & 7)
                px = plsc.load_gather(predb, [prow, lane * 0 + 1])
                py = plsc.load_gather(predb, [prow, lane * 0 + 2])
                pw = plsc.load_gather(predb, [prow, lane * 0 + 3])
                ph = plsc.load_gather(predb, [prow, lane * 0 + 4])
                sx = 1.0 / (1.0 + jnp.exp(-px))
                sy = 1.0 / (1.0 + jnp.exp(-py))
                dx, dy, dw, dh = sx - tx, sy - ty, pw - lw, ph - lh
                d = dx * dx + dy * dy + dw * dw + dh * dh
                acc = acc + jnp.where(vld, d, 0.0)
            return acc

        acc = lax.fori_loop(0, nb32, batch_body, acc)
        if nset == 1 and t + 1 < _NPW[i]:
            pending = issue_planes(t + 1)

    stage[...] = acc
    pltpu.sync_copy(stage, out_ref.at[wid, i])
    stage[...] = macc.astype(jnp.float32)
    pltpu.sync_copy(stage, out_ref.at[wid, 3 + i])


def _body(p0, p1, p2, t0, t1, t2, anch, out_ref,
          pa0, pb0, pc0, pd0, pe0,
          pa1, pb1, pc1, pd1, pe1,
          pa2, pb2, pc2, pd2, pe2,
          qa2, qb2, qc2, qd2, qe2,
          predb, ebuf, idx_cell, anch_v, stage, tsem0, tsem1, psem):
    wid = lax.axis_index("s") * _NC + lax.axis_index("c")
    pltpu.sync_copy(anch, anch_v)
    plane_sets = [
        [[pa0, pb0, pc0, pd0, pe0]],
        [[pa1, pb1, pc1, pd1, pe1]],
        [[pa2, pb2, pc2, pd2, pe2], [qa2, qb2, qc2, qd2, qe2]],
    ]
    preds = [p0, p1, p2]
    tgts = [t0, t1, t2]
    for i in range(3):
        _do_scale(i, preds[i], tgts[i], out_ref, plane_sets[i],
                  [tsem0, tsem1], predb, ebuf, idx_cell, anch_v, stage, psem,
                  wid)


_plane_shapes = [(13, 32), (32, 26), (32, 52)]

_sc_call = pl.kernel(
    _body,
    out_type=jax.ShapeDtypeStruct((_NW, 8, 16), jnp.float32),
    mesh=plsc.VectorSubcoreMesh(core_axis_name="c", subcore_axis_name="s",
                                num_cores=_NC, num_subcores=_NS),
    compiler_params=pltpu.CompilerParams(needs_layout_passes=False),
    scratch_types=(
        [pltpu.VMEM(_plane_shapes[0], jnp.float32) for _ in range(5)]
        + [pltpu.VMEM(_plane_shapes[1], jnp.float32) for _ in range(5)]
        + [pltpu.VMEM(_plane_shapes[2], jnp.float32) for _ in range(10)]
        + [
            pltpu.VMEM((256, 85), jnp.float32),
            pltpu.VMEM((32,), jnp.int32),
            pltpu.VMEM((2048,), jnp.int32),
            pltpu.VMEM((32,), jnp.float32),
            pltpu.VMEM((16,), jnp.float32),
            pltpu.SemaphoreType.DMA,
            pltpu.SemaphoreType.DMA,
            pltpu.SemaphoreType.DMA,
        ]
    ),
)


def kernel(pred_0, pred_1, pred_2, target_0, target_1, target_2):
    p = [x.transpose(1, 2, 3, 0, 4).reshape(-1, _BATCH, 85)
         for x in (pred_0, pred_1, pred_2)]
    t0 = target_0.transpose(1, 2, 4, 3, 0).reshape(-1, 6, 13, 32)
    t1 = target_1.transpose(1, 2, 4, 0, 3).reshape(-1, 6, 32, 26)
    t2 = target_2.transpose(1, 2, 4, 0, 3).reshape(-1, 6, 32, 52)
    anch = jnp.asarray(_anch_tab)
    parts = _sc_call(p[0], p[1], p[2], t0, t1, t2, anch)
    s = parts[:, 0:3, :].sum(axis=(0, 2))
    cnt = parts[:, 3:6, 0].sum(axis=0)
    return (10.0 * s / jnp.maximum(4.0 * cnt, 1.0)).sum()


# speedup vs baseline: 14.2140x; 1.0326x over previous
import numpy as np
import jax
import jax.numpy as jnp
from jax import lax
from jax.experimental import pallas as pl
from jax.experimental.pallas import tpu as pltpu
from jax.experimental.pallas import tpu_sc as plsc

_ANCHORS = np.array([
    [[0.28, 0.22], [0.38, 0.48], [0.90, 0.78]],
    [[0.07, 0.15], [0.15, 0.11], [0.14, 0.29]],
    [[0.02, 0.03], [0.04, 0.07], [0.08, 0.06]],
], dtype=np.float32)
_S_LIST = [13, 26, 52]
_BATCH = 32
_NC, _NS = 2, 16
_NW = _NC * _NS

_NBLK = [3 * s for s in _S_LIST]
_NPW = [(n + _NW - 1) // _NW for n in _NBLK]
_CB = [_BATCH * s for s in _S_LIST]
_NSET = [1, 1, 2]

_anch_tab = np.zeros(32, np.float32)
for _i in range(3):
    for _k in range(3):
        _anch_tab[_i * 8 + 2 * _k] = 1.0 / (_ANCHORS[_i, _k, 0] * _S_LIST[_i])
        _anch_tab[_i * 8 + 2 * _k + 1] = 1.0 / (_ANCHORS[_i, _k, 1] * _S_LIST[_i])

_LN2 = 0.6931471805599453
_SQRT2 = 1.4142135623730951


def _softlog(x):
    b = plsc.bitcast(x, jnp.int32)
    e = lax.shift_right_logical(b, 23) - 127
    m = plsc.bitcast((b & 0x007FFFFF) | 0x3F800000, jnp.float32)
    big = m >= _SQRT2
    m = jnp.where(big, m * 0.5, m)
    e = e + jnp.where(big, 1, 0)
    t = (m - 1.0) / (m + 1.0)
    t2 = t * t
    p = 1.0 + t2 * (1.0 / 3.0 + t2 * (0.2 + t2 * (1.0 / 7.0 + t2 * (1.0 / 9.0))))
    return e.astype(jnp.float32) * _LN2 + 2.0 * t * p


def _do_scale(i, pred_ref, tgt_ref, out_ref, plane_sets, tsems, predb,
              ebuf, idx_cell, anch_v, stage, psem, wid):
    s = _S_LIST[i]
    nblk, cb = _NBLK[i], _CB[i]
    nset = len(plane_sets)
    yb_order = (i == 0)
    lane = lax.iota(jnp.int32, 16)
    acc = jnp.zeros(16, jnp.float32)
    macc = jnp.zeros(16, jnp.int32)

    def issue_planes(t):
        blk = jnp.minimum(wid + t * _NW, nblk - 1)
        st = t % nset
        return [pltpu.async_copy(tgt_ref.at[blk, c], plane_sets[st][c],
                                 tsems[st]) for c in range(5)]

    pending = issue_planes(0)
    for t in range(_NPW[i]):
        planes = plane_sets[t % nset]
        blk_real = wid + t * _NW
        blk_ok = blk_real < nblk
        blk = jnp.minimum(blk_real, nblk - 1)

        for cp in pending:
            cp.wait()
        if nset > 1 and t + 1 < _NPW[i]:
            pending = issue_planes(t + 1)

        def scan_body(j, carry, blk_ok=blk_ok, planes=planes):
            m_vec, yv, bv = carry
            lidx = j * 16 + lane
            if yb_order:
                t0 = plsc.load_gather(planes[0], [yv, bv])
            else:
                t0 = plsc.load_gather(planes[0], [bv, yv])
            msk = (t0 == 1.0) & blk_ok
            plsc.store_compressed(idx_cell.at[pl.ds(m_vec[0], 16)], lidx,
                                  mask=msk)
            m_vec = m_vec + plsc.all_reduce_population_count(msk)
            if yb_order:
                yn = yv + (16 >> 5)
                bn = bv + 16
                wrap = bn >= 32
                bn = jnp.where(wrap, bn - 32, bn)
                yn = yv + jnp.where(wrap, 1, 0)
            else:
                yn = yv + 16
                wrap = yn >= s
                yn = jnp.where(wrap, yn - s, yn)
                bn = bv + jnp.where(wrap, 1, 0)
            return (m_vec, yn, bn)

        if yb_order:
            yinit = lax.shift_right_logical(lane, 5)
            binit = lane & 31
        else:
            binit = lane // s
            yinit = lane - binit * s
        m_vec, _, _ = lax.fori_loop(
            0, cb // 16, scan_body,
            (jnp.zeros(16, jnp.int32), yinit, binit))
        macc = macc + m_vec
        m = jnp.max(m_vec)
        nb32 = (m + 31) // 32

        def batch_body(v, acc, blk=blk, planes=planes):
            cells = []
            for w in range(2):
                cpos = v * 32 + w * 16 + lane
                vld = cpos < m
                cell = plsc.load_gather(idx_cell, [cpos])
                cell = jnp.where(vld, cell, 0)
                cells.append((cell, vld))
                for jj in range(16):
                    l_s = cell[jj]
                    if yb_order:
                        y_s = lax.shift_right_logical(l_s, 5)
                        b_s = l_s & 31
                    else:
                        b_s = l_s // s
                        y_s = l_s - b_s * s
                    b8 = pl.multiple_of(b_s & ~7, 8)
                    pltpu.async_copy(
                        pred_ref.at[blk * s + y_s, pl.ds(b8, 8)],
                        predb.at[pl.ds((w * 16 + jj) * 8, 8)], psem)
            tside = []
            for w in range(2):
                cell, vld = cells[w]
                if yb_order:
                    yv = lax.shift_right_logical(cell, 5)
                    bv = cell & 31
                    tx = plsc.load_gather(planes[1], [yv, bv])
                    ty = plsc.load_gather(planes[2], [yv, bv])
                    tw = plsc.load_gather(planes[3], [yv, bv])
                    th = plsc.load_gather(planes[4], [yv, bv])
                else:
                    bv = cell // s
                    yv = cell - bv * s
                    tx = plsc.load_gather(planes[1], [bv, yv])
                    ty = plsc.load_gather(planes[2], [bv, yv])
                    tw = plsc.load_gather(planes[3], [bv, yv])
                    th = plsc.load_gather(planes[4], [bv, yv])
                a_sc = blk // s
                iw = plsc.load_gather(anch_v, [lane * 0 + (i * 8 + 2 * a_sc)])
                ih = plsc.load_gather(anch_v,
                                      [lane * 0 + (i * 8 + 2 * a_sc + 1)])
                lw = _softlog(1e-16 + tw * iw)
                lh = _softlog(1e-16 + th * ih)
                tside.append((tx, ty, lw, lh, bv))
            for _ in range(8):
                pltpu.make_async_copy(pred_ref.at[0],
                                      predb.at[pl.ds(0, 32)], psem).wait()
            for w in range(2):
                cell, vld = cells[w]
                tx, ty, lw, lh, bv = tside[w]
                prow = (w * 16 + lane) * 8 + (bv & 7)
                px = plsc.load_gather(predb, [prow, lane * 0 + 1])
                py = plsc.load_gather(predb, [prow, lane * 0 + 2])
                pw = plsc.load_gather(predb, [prow, lane * 0 + 3])
                ph = plsc.load_gather(predb, [prow, lane * 0 + 4])
                sx = 1.0 / (1.0 + jnp.exp(-px))
                sy = 1.0 / (1.0 + jnp.exp(-py))
                dx, dy, dw, dh = sx - tx, sy - ty, pw - lw, ph - lh
                d = dx * dx + dy * dy + dw * dw + dh * dh
                acc = acc + jnp.where(vld, d, 0.0)
            return acc

        acc = lax.fori_loop(0, nb32, batch_body, acc)
        if nset == 1 and t + 1 < _NPW[i]:
            pending = issue_planes(t + 1)

    stage[...] = acc
    pltpu.sync_copy(stage, out_ref.at[wid, i])
    stage[...] = macc.astype(jnp.float32)
    pltpu.sync_copy(stage, out_ref.at[wid, 3 + i])


def _body(p0, p1, p2, t0, t1, t2, anch, out_ref,
          pa0, pb0, pc0, pd0, pe0,
          pa1, pb1, pc1, pd1, pe1,
          pa2, pb2, pc2, pd2, pe2,
          qa2, qb2, qc2, qd2, qe2,
          predb, ebuf, idx_cell, anch_v, stage, tsem0, tsem1, psem):
    wid = lax.axis_index("s") * _NC + lax.axis_index("c")
    pltpu.sync_copy(anch, anch_v)
    plane_sets = [
        [[pa0, pb0, pc0, pd0, pe0]],
        [[pa1, pb1, pc1, pd1, pe1]],
        [[pa2, pb2, pc2, pd2, pe2], [qa2, qb2, qc2, qd2, qe2]],
    ]
    preds = [p0, p1, p2]
    tgts = [t0, t1, t2]
    for i in range(3):
        _do_scale(i, preds[i], tgts[i], out_ref, plane_sets[i],
                  [tsem0, tsem1], predb, ebuf, idx_cell, anch_v, stage, psem,
                  wid)


_plane_shapes = [(13, 32), (32, 26), (32, 52)]

_sc_call = pl.kernel(
    _body,
    out_type=jax.ShapeDtypeStruct((_NW, 8, 16), jnp.float32),
    mesh=plsc.VectorSubcoreMesh(core_axis_name="c", subcore_axis_name="s",
                                num_cores=_NC, num_subcores=_NS),
    compiler_params=pltpu.CompilerParams(needs_layout_passes=False),
    scratch_types=(
        [pltpu.VMEM(_plane_shapes[0], jnp.float32) for _ in range(5)]
        + [pltpu.VMEM(_plane_shapes[1], jnp.float32) for _ in range(5)]
        + [pltpu.VMEM(_plane_shapes[2], jnp.float32) for _ in range(10)]
        + [
            pltpu.VMEM((256, 85), jnp.float32),
            pltpu.VMEM((32,), jnp.int32),
            pltpu.VMEM((2048,), jnp.int32),
            pltpu.VMEM((32,), jnp.float32),
            pltpu.VMEM((16,), jnp.float32),
            pltpu.SemaphoreType.DMA,
            pltpu.SemaphoreType.DMA,
            pltpu.SemaphoreType.DMA,
        ]
    ),
)


def kernel(pred_0, pred_1, pred_2, target_0, target_1, target_2):
    p = [x.transpose(1, 2, 3, 0, 4).reshape(-1, _BATCH, 85)
         for x in (pred_0, pred_1, pred_2)]
    t0 = target_0.transpose(1, 2, 4, 3, 0).reshape(-1, 6, 13, 32)
    t1 = target_1.transpose(1, 2, 4, 0, 3).reshape(-1, 6, 32, 26)
    t2 = target_2.transpose(1, 2, 4, 0, 3).reshape(-1, 6, 32, 52)
    anch = jnp.asarray(_anch_tab)
    parts = _sc_call(p[0], p[1], p[2], t0, t1, t2, anch)
    s = parts[:, 0:3, :].sum(axis=(0, 2))
    cnt = parts[:, 3:6, 0].sum(axis=0)
    return (10.0 * s / jnp.maximum(4.0 * cnt, 1.0)).sum()
